# baseline (speedup 1.0000x reference)
import jax
import jax.numpy as jnp
from jax import lax
from jax.experimental import pallas as pl
from jax.experimental.pallas import tpu as pltpu
from jax.experimental.pallas import tpu_sc as plsc

N = 10000
D = 128
E = 320000
NC = 2
NS = 16
NW = NC * NS
CHUNK = 128
NCH = 80
EPT = NCH * CHUNK
EP = NW * EPT
NP = 10240
CHUNK_C = 96
NCHA0, NCHA1 = 112, 48
NCHC0, NCHC1 = 152, 60
EP_C = NS * (NCHC0 + NCHC1) * CHUNK_C
NEG = -1e30


def _iota16():
    return lax.iota(jnp.int32, 16)


def _lanesum(v):
    i = _iota16()
    for sh in (8, 4, 2, 1):
        v = v + v.at[i ^ sh].get(mode="promise_in_bounds")
    return v


def _s0_body(x_ref, w_ref, b_ref, h_ref, hl_ref):
    h = jnp.dot(x_ref[...], w_ref[...], preferred_element_type=jnp.float32)
    h = h + b_ref[...]
    h_ref[...] = h
    col = lax.broadcasted_iota(jnp.int32, h.shape, 1)
    hl_ref[...] = jnp.where(col == 0, -h, h)


def _transform(x, wt, bt):
    blk = 1000
    return pl.pallas_call(
        _s0_body,
        grid=(N // blk,),
        in_specs=[
            pl.BlockSpec((blk, D), lambda i: (i, 0)),
            pl.BlockSpec((D, D), lambda i: (0, 0)),
            pl.BlockSpec((1, D), lambda i: (0, 0)),
        ],
        out_specs=[
            pl.BlockSpec((blk, D), lambda i: (i, 0)),
            pl.BlockSpec((blk, D), lambda i: (i, 0)),
        ],
        out_shape=[
            jax.ShapeDtypeStruct((N, D), jnp.float32),
            jax.ShapeDtypeStruct((N, D), jnp.float32),
        ],
    )(x, wt, bt.reshape(1, D))


def _stage_a_body(hl_hbm, h_hbm, pidx_hbm, att_hbm,
                  scores_hbm, mpart_hbm,
                  att_v, m_v, pidx0, pidx1, hsl0, hsl1, hd0, hd1, sc_v,
                  sema0, sema1, semb0, semb1):
    cid = lax.axis_index("c")
    sid = lax.axis_index("s")
    wid = cid * NS + sid
    nch = jnp.where(cid == 0, NCHA0, NCHA1)
    cbase = jnp.where(cid == 0, sid * NCHA0, NS * NCHA0 + sid * NCHA1)
    pidx = (pidx0, pidx1)
    hsl = (hsl0, hsl1)
    hd = (hd0, hd1)
    sema = (sema0, sema1)
    semb = (semb0, semb1)
    pltpu.sync_copy(att_hbm, att_v)

    def init_m(i, _):
        m_v[pl.ds(i * 16, 16)] = jnp.full((16,), NEG, jnp.float32)
        return 0
    lax.fori_loop(0, N // 16, init_m, 0)

    def fetch(nci, q):
        pltpu.sync_copy(pidx_hbm.at[cbase + nci], pidx[q])
        pltpu.async_copy(hl_hbm.at[pidx[q].at[0]], hsl[q], sema[q])
        pltpu.async_copy(h_hbm.at[pidx[q].at[1]], hd[q], semb[q])

    def compute(ci, p):
        base = (cbase + ci) * CHUNK

        def group_body(g, _):
            gbase = g * 16
            lor = jnp.zeros((16,), jnp.float32)
            for j in range(16):
                acc = jnp.zeros((16,), jnp.float32)
                for k in range(D // 32):
                    a = plsc.bitcast(hsl[p][gbase + j, pl.ds(k * 16, 16)],
                                     jnp.bfloat16)
                    b = plsc.bitcast(hd[p][gbase + j, pl.ds(k * 16, 16)],
                                     jnp.bfloat16)
                    a0, a1 = plsc.unpack(a, format=plsc.PackFormat.INTERLEAVED)
                    b0, b1 = plsc.unpack(b, format=plsc.PackFormat.INTERLEAVED)
                    acc = acc + a0 * b0 + a1 * b1
                dj = _lanesum(acc)
                lor = jnp.where(_iota16() == j, dj, lor)

            src_g = pidx[p][0, pl.ds(gbase, 16)]
            dst_g = pidx[p][1, pl.ds(gbase, 16)]
            adj_g = plsc.bitcast(pidx[p][2, pl.ds(gbase, 16)], jnp.float32)
            att_s = plsc.load_gather(att_v, [src_g])
            att_d = plsc.load_gather(att_v, [dst_g])
            s = adj_g * lor + att_s + att_d
            s = jnp.where(s >= 0.0, s, 0.2 * s)
            eid = base + gbase + _iota16()
            s = jnp.where(eid < E, s, NEG)
            sc_v[pl.ds(gbase, 16)] = s

            cur = plsc.load_gather(m_v, [dst_g])

            def cond(carry):
                return carry[1]

            def body(carry):
                c2 = plsc.load_gather(m_v, [dst_g])
                plsc.store_scatter(m_v, [dst_g], jnp.maximum(c2, s),
                                   mask=s > c2)
                c3 = plsc.load_gather(m_v, [dst_g])
                return (c3, jnp.any(s > c3))
            lax.while_loop(cond, body, (cur, jnp.any(s > cur)))
            return 0
        lax.fori_loop(0, CHUNK // 16, group_body, 0)
        pltpu.sync_copy(sc_v, scores_hbm.at[pl.ds(base, CHUNK)])

    fetch(0, 0)

    def pair_body(t, _):
        for half in (0, 1):
            pb = half
            qb = 1 - half
            ci = t * 2 + half
            nci = ci + 1

            @pl.when(nci < nch)
            def _():
                fetch(nci, qb)
            pltpu.make_async_copy(
                hl_hbm.at[pidx[pb].at[0]], hsl[pb], sema[pb]).wait()
            pltpu.make_async_copy(
                h_hbm.at[pidx[pb].at[1]], hd[pb], semb[pb]).wait()
            compute(ci, pb)
        return 0
    lax.fori_loop(0, nch // 2, pair_body, 0)
    pltpu.sync_copy(m_v, mpart_hbm.at[wid])


def _stage_a(hl, h, pidx, att):
    mesh = plsc.VectorSubcoreMesh(core_axis_name="c", subcore_axis_name="s")
    return pl.kernel(
        _stage_a_body,
        out_type=[
            jax.ShapeDtypeStruct((EP,), jnp.float32),
            jax.ShapeDtypeStruct((NW, N), jnp.float32),
        ],
        mesh=mesh,
        compiler_params=pltpu.CompilerParams(needs_layout_passes=False,
                                             use_tc_tiling_on_sc=False),
        scratch_types=[
            pltpu.VMEM((N,), jnp.float32),
            pltpu.VMEM((N,), jnp.float32),
            pltpu.VMEM((3, CHUNK), jnp.int32),
            pltpu.VMEM((3, CHUNK), jnp.int32),
            pltpu.VMEM((CHUNK, D // 2), jnp.int32),
            pltpu.VMEM((CHUNK, D // 2), jnp.int32),
            pltpu.VMEM((CHUNK, D // 2), jnp.int32),
            pltpu.VMEM((CHUNK, D // 2), jnp.int32),
            pltpu.VMEM((CHUNK,), jnp.float32),
            pltpu.SemaphoreType.DMA,
            pltpu.SemaphoreType.DMA,
            pltpu.SemaphoreType.DMA,
            pltpu.SemaphoreType.DMA,
        ],
    )(hl, h, pidx, att)


def _merge_max_body(mp_ref, m_ref):
    m_ref[...] = jnp.max(mp_ref[...], axis=0, keepdims=True)


def _merge_max(mpart):
    return pl.pallas_call(
        _merge_max_body,
        out_shape=jax.ShapeDtypeStruct((1, N), jnp.float32),
    )(mpart)


def _seg_add(denom_ref, dst, vals):
    d_s, e_s = plsc.sort_key_val(dst, vals)
    c = plsc.cumsum(e_s)
    i = _iota16()
    d_next = d_s.at[jnp.minimum(i + 1, 15)].get(mode="promise_in_bounds")
    last = (i == 15) | (d_s != d_next)
    d_prev = d_s.at[jnp.maximum(i - 1, 0)].get(mode="promise_in_bounds")
    first = (i == 0) | (d_s != d_prev)
    pc = c.at[jnp.maximum(i - 1, 0)].get(mode="promise_in_bounds")
    pc = jnp.where(i == 0, 0.0, pc)
    base = jnp.where(first, pc, -1.0)
    baseprop = plsc.cummax(base)
    seg = c - baseprop
    cur = plsc.load_gather(denom_ref, [d_s])
    plsc.store_scatter(denom_ref, [d_s], cur + seg, mask=last)


def _stage_c_body(h_hbm, pidx_hbm, scores_hbm, m_hbm,
                  denom_hbm, msgpart_hbm,
                  m_v, pidx0, pidx1, sc0, sc1, e_v,
                  g0, g1, msg_sh, den_sh,
                  semg0, semg1):
    cid = lax.axis_index("c")
    sid = lax.axis_index("s")
    wid = cid * NS + sid
    nch = jnp.where(cid == 0, NCHC0, NCHC1)
    cbase = jnp.where(cid == 0, sid * NCHC0, NS * NCHC0 + sid * NCHC1)
    pidx = (pidx0, pidx1)
    scv = (sc0, sc1)
    gv = (g0, g1)
    semg = (semg0, semg1)
    pltpu.sync_copy(m_hbm, m_v)

    def init_f(i, _):
        for k in range(D // 16):
            g0[i, pl.ds(k * 16, 16)] = jnp.zeros((16,), jnp.float32)
        return 0
    lax.fori_loop(0, CHUNK_C, init_f, 0)

    def init_e(i, _):
        e_v[pl.ds(i * 16, 16)] = jnp.zeros((16,), jnp.float32)
        return 0
    lax.fori_loop(0, CHUNK_C // 16, init_e, 0)
    rows_per_tile = NP // NS
    r0 = sid * rows_per_tile
    for z in range(6):
        pltpu.sync_copy(g0, msg_sh.at[pl.ds(r0 + z * CHUNK_C, CHUNK_C)])
        pltpu.sync_copy(e_v, den_sh.at[pl.ds(r0 + z * CHUNK_C, CHUNK_C)])
    pltpu.sync_copy(g0.at[pl.ds(0, 64)],
                    msg_sh.at[pl.ds(r0 + 6 * CHUNK_C, 64)])
    pltpu.sync_copy(e_v.at[pl.ds(0, 64)],
                    den_sh.at[pl.ds(r0 + 6 * CHUNK_C, 64)])
    plsc.subcore_barrier()

    def fetch(nci, q):
        pltpu.sync_copy(pidx_hbm.at[cbase + nci], pidx[q])
        pltpu.sync_copy(
            scores_hbm.at[pl.ds((cbase + nci) * CHUNK_C, CHUNK_C)], scv[q])
        pltpu.async_copy(h_hbm.at[pidx[q].at[0]], gv[q], semg[q])

    def compute(ci, p):
        def group_body(g, _):
            gbase = g * 16
            s = scv[p][pl.ds(gbase, 16)]
            dst_g = pidx[p][1, pl.ds(gbase, 16)]
            m_d = plsc.load_gather(m_v, [dst_g])
            e = jnp.where(s < -5e29, 0.0, jnp.exp(s - m_d))
            e_v[pl.ds(gbase, 16)] = e
            return 0
        lax.fori_loop(0, CHUNK_C // 16, group_body, 0)
        pltpu.sync_copy(e_v, den_sh.at[pidx[p].at[1]], add=True)

        def scale_body(g, _):
            e16 = e_v[pl.ds(g * 16, 16)]
            for j in range(16):
                a = e16[j]
                r = g * 16 + j
                for k in range(D // 16):
                    gv[p][r, pl.ds(k * 16, 16)] = (
                        gv[p][r, pl.ds(k * 16, 16)] * a)
            return 0
        lax.fori_loop(0, CHUNK_C // 16, scale_body, 0)
        pltpu.sync_copy(gv[p], msg_sh.at[pidx[p].at[1]], add=True)

    fetch(0, 0)

    def pair_body(t, _):
        for half in (0, 1):
            pb = half
            qb = 1 - half
            ci = t * 2 + half
            nci = ci + 1

            @pl.when(nci < nch)
            def _():
                fetch(nci, qb)
            pltpu.make_async_copy(
                h_hbm.at[pidx[pb].at[0]], gv[pb], semg[pb]).wait()
            compute(ci, pb)
        return 0
    lax.fori_loop(0, nch // 2, pair_body, 0)
    plsc.subcore_barrier()
    pltpu.sync_copy(den_sh.at[pl.ds(r0, rows_per_tile)],
                    denom_hbm.at[cid, pl.ds(r0, rows_per_tile)])
    pltpu.sync_copy(msg_sh.at[pl.ds(r0, rows_per_tile)],
                    msgpart_hbm.at[cid, pl.ds(r0, rows_per_tile)])


def _stage_c(h, pidx, scores, m):
    mesh = plsc.VectorSubcoreMesh(core_axis_name="c", subcore_axis_name="s")
    return pl.kernel(
        _stage_c_body,
        out_type=[
            jax.ShapeDtypeStruct((NC, NP), jnp.float32),
            jax.ShapeDtypeStruct((NC, NP, D), jnp.float32),
        ],
        mesh=mesh,
        compiler_params=pltpu.CompilerParams(needs_layout_passes=False,
                                             use_tc_tiling_on_sc=False),
        scratch_types=[
            pltpu.VMEM((N,), jnp.float32),
            pltpu.VMEM((2, CHUNK_C), jnp.int32),
            pltpu.VMEM((2, CHUNK_C), jnp.int32),
            pltpu.VMEM((CHUNK_C,), jnp.float32),
            pltpu.VMEM((CHUNK_C,), jnp.float32),
            pltpu.VMEM((CHUNK_C,), jnp.float32),
            pltpu.VMEM((CHUNK_C, D), jnp.float32),
            pltpu.VMEM((CHUNK_C, D), jnp.float32),
            pltpu.VMEM_SHARED((NP, D), jnp.float32),
            pltpu.VMEM_SHARED((NP,), jnp.float32),
            pltpu.SemaphoreType.DMA,
            pltpu.SemaphoreType.DMA,
        ],
    )(h, pidx, scores, m)


def _d_body(msg0_ref, msg1_ref, den_ref, wa_ref, ba_ref, wo_ref, bo_ref,
            out_ref):
    dsum = jnp.sum(den_ref[...], axis=1, keepdims=True) + 1e-9
    msg = (msg0_ref[...] + msg1_ref[...]) * (1.0 / dsum)
    act = jnp.tanh(
        jnp.dot(msg, wa_ref[...], preferred_element_type=jnp.float32)
        + ba_ref[...])
    out_ref[...] = (
        jnp.dot(act, wo_ref[...], preferred_element_type=jnp.float32)
        + bo_ref[...])


def _stage_d(msgpart, denom, wa, ba, wo, bo):
    blk = 1000
    return pl.pallas_call(
        _d_body,
        grid=(N // blk,),
        in_specs=[
            pl.BlockSpec((blk, D), lambda i: (i, 0)),
            pl.BlockSpec((blk, D), lambda i: (i, 0)),
            pl.BlockSpec((blk, NC), lambda i: (i, 0)),
            pl.BlockSpec((D, D), lambda i: (0, 0)),
            pl.BlockSpec((1, D), lambda i: (0, 0)),
            pl.BlockSpec((D, D), lambda i: (0, 0)),
            pl.BlockSpec((1, D), lambda i: (0, 0)),
        ],
        out_specs=pl.BlockSpec((blk, D), lambda i: (i, 0)),
        out_shape=jax.ShapeDtypeStruct((N, D), jnp.float32),
    )(msgpart[0], msgpart[1], denom.T, wa, ba.reshape(1, D), wo,
      bo.reshape(1, D))


def kernel(node_features, adj_indices, adj_values, adj_dense_shape,
           attention_weights, transform_weights, transform_bias,
           activation_weights, activation_bias, output_weights, output_bias):
    x = node_features[0]
    att = attention_weights[0, :, 0]
    src = adj_indices[:, 1]
    dst = adj_indices[:, 2]
    pad = EP - E
    src_p = jnp.pad(src, (0, pad))
    dst_p = jnp.pad(dst, (0, pad))
    adj_p = jnp.pad(adj_values, (0, pad))
    adj_b = lax.bitcast_convert_type(adj_p, jnp.int32)
    pidx = jnp.stack([src_p, dst_p, adj_b], axis=0)
    pidx_a = pidx.reshape(3, NW * NCH, CHUNK).transpose(1, 0, 2)
    pidx_c = pidx[:2, :EP_C].reshape(2, EP_C // CHUNK_C, CHUNK_C)
    pidx_c = pidx_c.transpose(1, 0, 2)

    h, hl = _transform(x, transform_weights, transform_bias)

    def packb(t):
        tb = t.astype(jnp.bfloat16).reshape(N, D // 32, 2, 16)
        tb = tb.transpose(0, 1, 3, 2)
        return lax.bitcast_convert_type(tb, jnp.int32).reshape(N, D // 2)

    hlb = packb(hl)
    hb = packb(h)
    scores, mpart = _stage_a(hlb, hb, pidx_a, att)
    m = _merge_max(mpart).reshape(N)
    denom, msgpart = _stage_c(h, pidx_c, scores, m)
    denom = denom[:, :N]
    msgpart = msgpart[:, :N, :]
    out = _stage_d(msgpart, denom, activation_weights, activation_bias,
                   output_weights, output_bias)
    return out[None, :, :]


# speedup vs baseline: 1.0209x; 1.0209x over previous
import jax
import jax.numpy as jnp
from jax import lax
from jax.experimental import pallas as pl
from jax.experimental.pallas import tpu as pltpu
from jax.experimental.pallas import tpu_sc as plsc

N = 10000
D = 128
E = 320000
NC = 2
NS = 16
NW = NC * NS
CHUNK = 128
NCH = 80
EPT = NCH * CHUNK
EP = NW * EPT
NP = 10240
CHUNK_C = 96
NCHA0, NCHA1 = 112, 48
NCHC0, NCHC1 = 140, 72
EP_C = NS * (NCHC0 + NCHC1) * CHUNK_C
NEG = -1e30


def _iota16():
    return lax.iota(jnp.int32, 16)


def _lanesum(v):
    i = _iota16()
    for sh in (8, 4, 2, 1):
        v = v + v.at[i ^ sh].get(mode="promise_in_bounds")
    return v


def _s0_body(x_ref, w_ref, b_ref, h_ref, hl_ref):
    h = jnp.dot(x_ref[...], w_ref[...], preferred_element_type=jnp.float32)
    h = h + b_ref[...]
    h_ref[...] = h
    col = lax.broadcasted_iota(jnp.int32, h.shape, 1)
    hl_ref[...] = jnp.where(col == 0, -h, h)


def _transform(x, wt, bt):
    blk = 1000
    return pl.pallas_call(
        _s0_body,
        grid=(N // blk,),
        in_specs=[
            pl.BlockSpec((blk, D), lambda i: (i, 0)),
            pl.BlockSpec((D, D), lambda i: (0, 0)),
            pl.BlockSpec((1, D), lambda i: (0, 0)),
        ],
        out_specs=[
            pl.BlockSpec((blk, D), lambda i: (i, 0)),
            pl.BlockSpec((blk, D), lambda i: (i, 0)),
        ],
        out_shape=[
            jax.ShapeDtypeStruct((N, D), jnp.float32),
            jax.ShapeDtypeStruct((N, D), jnp.float32),
        ],
    )(x, wt, bt.reshape(1, D))


def _stage_a_body(hl_hbm, h_hbm, pidx_hbm, att_hbm,
                  scores_hbm, mpart_hbm,
                  att_v, m_v, pidx0, pidx1, hsl0, hsl1, hd0, hd1, sc_v,
                  sema0, sema1, semb0, semb1):
    cid = lax.axis_index("c")
    sid = lax.axis_index("s")
    wid = cid * NS + sid
    nch = jnp.where(cid == 0, NCHA0, NCHA1)
    cbase = jnp.where(cid == 0, sid * NCHA0, NS * NCHA0 + sid * NCHA1)
    pidx = (pidx0, pidx1)
    hsl = (hsl0, hsl1)
    hd = (hd0, hd1)
    sema = (sema0, sema1)
    semb = (semb0, semb1)
    pltpu.sync_copy(att_hbm, att_v)

    def init_m(i, _):
        m_v[pl.ds(i * 16, 16)] = jnp.full((16,), NEG, jnp.float32)
        return 0
    lax.fori_loop(0, N // 16, init_m, 0)

    def fetch(nci, q):
        pltpu.sync_copy(pidx_hbm.at[cbase + nci], pidx[q])
        pltpu.async_copy(hl_hbm.at[pidx[q].at[0]], hsl[q], sema[q])
        pltpu.async_copy(h_hbm.at[pidx[q].at[1]], hd[q], semb[q])

    def compute(ci, p):
        base = (cbase + ci) * CHUNK

        def group_body(g, _):
            gbase = g * 16
            lor = jnp.zeros((16,), jnp.float32)
            for j in range(16):
                acc = jnp.zeros((16,), jnp.float32)
                for k in range(D // 32):
                    a = plsc.bitcast(hsl[p][gbase + j, pl.ds(k * 16, 16)],
                                     jnp.bfloat16)
                    b = plsc.bitcast(hd[p][gbase + j, pl.ds(k * 16, 16)],
                                     jnp.bfloat16)
                    a0, a1 = plsc.unpack(a, format=plsc.PackFormat.INTERLEAVED)
                    b0, b1 = plsc.unpack(b, format=plsc.PackFormat.INTERLEAVED)
                    acc = acc + a0 * b0 + a1 * b1
                dj = _lanesum(acc)
                lor = jnp.where(_iota16() == j, dj, lor)

            src_g = pidx[p][0, pl.ds(gbase, 16)]
            dst_g = pidx[p][1, pl.ds(gbase, 16)]
            adj_g = plsc.bitcast(pidx[p][2, pl.ds(gbase, 16)], jnp.float32)
            att_s = plsc.load_gather(att_v, [src_g])
            att_d = plsc.load_gather(att_v, [dst_g])
            s = adj_g * lor + att_s + att_d
            s = jnp.where(s >= 0.0, s, 0.2 * s)
            eid = base + gbase + _iota16()
            s = jnp.where(eid < E, s, NEG)
            sc_v[pl.ds(gbase, 16)] = s

            cur = plsc.load_gather(m_v, [dst_g])

            def cond(carry):
                return carry[1]

            def body(carry):
                c2 = plsc.load_gather(m_v, [dst_g])
                plsc.store_scatter(m_v, [dst_g], jnp.maximum(c2, s),
                                   mask=s > c2)
                c3 = plsc.load_gather(m_v, [dst_g])
                return (c3, jnp.any(s > c3))
            lax.while_loop(cond, body, (cur, jnp.any(s > cur)))
            return 0
        lax.fori_loop(0, CHUNK // 16, group_body, 0)
        pltpu.sync_copy(sc_v, scores_hbm.at[pl.ds(base, CHUNK)])

    fetch(0, 0)

    def pair_body(t, _):
        for half in (0, 1):
            pb = half
            qb = 1 - half
            ci = t * 2 + half
            nci = ci + 1

            @pl.when(nci < nch)
            def _():
                fetch(nci, qb)
            pltpu.make_async_copy(
                hl_hbm.at[pidx[pb].at[0]], hsl[pb], sema[pb]).wait()
            pltpu.make_async_copy(
                h_hbm.at[pidx[pb].at[1]], hd[pb], semb[pb]).wait()
            compute(ci, pb)
        return 0
    lax.fori_loop(0, nch // 2, pair_body, 0)
    pltpu.sync_copy(m_v, mpart_hbm.at[wid])


def _stage_a(hl, h, pidx, att):
    mesh = plsc.VectorSubcoreMesh(core_axis_name="c", subcore_axis_name="s")
    return pl.kernel(
        _stage_a_body,
        out_type=[
            jax.ShapeDtypeStruct((EP,), jnp.float32),
            jax.ShapeDtypeStruct((NW, N), jnp.float32),
        ],
        mesh=mesh,
        compiler_params=pltpu.CompilerParams(needs_layout_passes=False,
                                             use_tc_tiling_on_sc=False),
        scratch_types=[
            pltpu.VMEM((N,), jnp.float32),
            pltpu.VMEM((N,), jnp.float32),
            pltpu.VMEM((3, CHUNK), jnp.int32),
            pltpu.VMEM((3, CHUNK), jnp.int32),
            pltpu.VMEM((CHUNK, D // 2), jnp.int32),
            pltpu.VMEM((CHUNK, D // 2), jnp.int32),
            pltpu.VMEM((CHUNK, D // 2), jnp.int32),
            pltpu.VMEM((CHUNK, D // 2), jnp.int32),
            pltpu.VMEM((CHUNK,), jnp.float32),
            pltpu.SemaphoreType.DMA,
            pltpu.SemaphoreType.DMA,
            pltpu.SemaphoreType.DMA,
            pltpu.SemaphoreType.DMA,
        ],
    )(hl, h, pidx, att)


def _merge_max_body(mp_ref, m_ref):
    m_ref[...] = jnp.max(mp_ref[...], axis=0, keepdims=True)


def _merge_max(mpart):
    return pl.pallas_call(
        _merge_max_body,
        out_shape=jax.ShapeDtypeStruct((1, N), jnp.float32),
    )(mpart)


def _seg_add(denom_ref, dst, vals):
    d_s, e_s = plsc.sort_key_val(dst, vals)
    c = plsc.cumsum(e_s)
    i = _iota16()
    d_next = d_s.at[jnp.minimum(i + 1, 15)].get(mode="promise_in_bounds")
    last = (i == 15) | (d_s != d_next)
    d_prev = d_s.at[jnp.maximum(i - 1, 0)].get(mode="promise_in_bounds")
    first = (i == 0) | (d_s != d_prev)
    pc = c.at[jnp.maximum(i - 1, 0)].get(mode="promise_in_bounds")
    pc = jnp.where(i == 0, 0.0, pc)
    base = jnp.where(first, pc, -1.0)
    baseprop = plsc.cummax(base)
    seg = c - baseprop
    cur = plsc.load_gather(denom_ref, [d_s])
    plsc.store_scatter(denom_ref, [d_s], cur + seg, mask=last)


def _stage_c_body(h_hbm, pidx_hbm, scores_hbm, m_hbm,
                  denom_hbm, msgpart_hbm,
                  m_v, pidx0, pidx1, sc0, sc1, e_v,
                  g0, g1, msg_sh, den_sh,
                  semg0, semg1):
    cid = lax.axis_index("c")
    sid = lax.axis_index("s")
    wid = cid * NS + sid
    nch = jnp.where(cid == 0, NCHC0, NCHC1)
    cbase = jnp.where(cid == 0, sid * NCHC0, NS * NCHC0 + sid * NCHC1)
    pidx = (pidx0, pidx1)
    scv = (sc0, sc1)
    gv = (g0, g1)
    semg = (semg0, semg1)
    pltpu.sync_copy(m_hbm, m_v)

    def init_f(i, _):
        for k in range(D // 16):
            g0[i, pl.ds(k * 16, 16)] = jnp.zeros((16,), jnp.float32)
        return 0
    lax.fori_loop(0, CHUNK_C, init_f, 0)

    def init_e(i, _):
        e_v[pl.ds(i * 16, 16)] = jnp.zeros((16,), jnp.float32)
        return 0
    lax.fori_loop(0, CHUNK_C // 16, init_e, 0)
    rows_per_tile = NP // NS
    r0 = sid * rows_per_tile
    for z in range(6):
        pltpu.sync_copy(g0, msg_sh.at[pl.ds(r0 + z * CHUNK_C, CHUNK_C)])
        pltpu.sync_copy(e_v, den_sh.at[pl.ds(r0 + z * CHUNK_C, CHUNK_C)])
    pltpu.sync_copy(g0.at[pl.ds(0, 64)],
                    msg_sh.at[pl.ds(r0 + 6 * CHUNK_C, 64)])
    pltpu.sync_copy(e_v.at[pl.ds(0, 64)],
                    den_sh.at[pl.ds(r0 + 6 * CHUNK_C, 64)])
    plsc.subcore_barrier()

    def fetch(nci, q):
        pltpu.sync_copy(pidx_hbm.at[cbase + nci], pidx[q])
        pltpu.sync_copy(
            scores_hbm.at[pl.ds((cbase + nci) * CHUNK_C, CHUNK_C)], scv[q])
        pltpu.async_copy(h_hbm.at[pidx[q].at[0]], gv[q], semg[q])

    def compute(ci, p):
        def group_body(g, _):
            gbase = g * 16
            s = scv[p][pl.ds(gbase, 16)]
            dst_g = pidx[p][1, pl.ds(gbase, 16)]
            m_d = plsc.load_gather(m_v, [dst_g])
            e = jnp.where(s < -5e29, 0.0, jnp.exp(s - m_d))
            e_v[pl.ds(gbase, 16)] = e
            return 0
        lax.fori_loop(0, CHUNK_C // 16, group_body, 0)
        pltpu.sync_copy(e_v, den_sh.at[pidx[p].at[1]], add=True)

        def scale_body(g, _):
            e16 = e_v[pl.ds(g * 16, 16)]
            for j in range(16):
                a = e16[j]
                r = g * 16 + j
                for k in range(D // 16):
                    gv[p][r, pl.ds(k * 16, 16)] = (
                        gv[p][r, pl.ds(k * 16, 16)] * a)
            return 0
        lax.fori_loop(0, CHUNK_C // 16, scale_body, 0)
        pltpu.sync_copy(gv[p], msg_sh.at[pidx[p].at[1]], add=True)

    fetch(0, 0)

    def pair_body(t, _):
        for half in (0, 1):
            pb = half
            qb = 1 - half
            ci = t * 2 + half
            nci = ci + 1

            @pl.when(nci < nch)
            def _():
                fetch(nci, qb)
            pltpu.make_async_copy(
                h_hbm.at[pidx[pb].at[0]], gv[pb], semg[pb]).wait()
            compute(ci, pb)
        return 0
    lax.fori_loop(0, nch // 2, pair_body, 0)
    plsc.subcore_barrier()
    pltpu.sync_copy(den_sh.at[pl.ds(r0, rows_per_tile)],
                    denom_hbm.at[cid, pl.ds(r0, rows_per_tile)])
    pltpu.sync_copy(msg_sh.at[pl.ds(r0, rows_per_tile)],
                    msgpart_hbm.at[cid, pl.ds(r0, rows_per_tile)])


def _stage_c(h, pidx, scores, m):
    mesh = plsc.VectorSubcoreMesh(core_axis_name="c", subcore_axis_name="s")
    return pl.kernel(
        _stage_c_body,
        out_type=[
            jax.ShapeDtypeStruct((NC, NP), jnp.float32),
            jax.ShapeDtypeStruct((NC, NP, D), jnp.float32),
        ],
        mesh=mesh,
        compiler_params=pltpu.CompilerParams(needs_layout_passes=False,
                                             use_tc_tiling_on_sc=False),
        scratch_types=[
            pltpu.VMEM((N,), jnp.float32),
            pltpu.VMEM((2, CHUNK_C), jnp.int32),
            pltpu.VMEM((2, CHUNK_C), jnp.int32),
            pltpu.VMEM((CHUNK_C,), jnp.float32),
            pltpu.VMEM((CHUNK_C,), jnp.float32),
            pltpu.VMEM((CHUNK_C,), jnp.float32),
            pltpu.VMEM((CHUNK_C, D), jnp.float32),
            pltpu.VMEM((CHUNK_C, D), jnp.float32),
            pltpu.VMEM_SHARED((NP, D), jnp.float32),
            pltpu.VMEM_SHARED((NP,), jnp.float32),
            pltpu.SemaphoreType.DMA,
            pltpu.SemaphoreType.DMA,
        ],
    )(h, pidx, scores, m)


def _d_body(msg0_ref, msg1_ref, den_ref, wa_ref, ba_ref, wo_ref, bo_ref,
            out_ref):
    dsum = jnp.sum(den_ref[...], axis=1, keepdims=True) + 1e-9
    msg = (msg0_ref[...] + msg1_ref[...]) * (1.0 / dsum)
    act = jnp.tanh(
        jnp.dot(msg, wa_ref[...], preferred_element_type=jnp.float32)
        + ba_ref[...])
    out_ref[...] = (
        jnp.dot(act, wo_ref[...], preferred_element_type=jnp.float32)
        + bo_ref[...])


def _stage_d(msgpart, denom, wa, ba, wo, bo):
    blk = 1000
    return pl.pallas_call(
        _d_body,
        grid=(N // blk,),
        in_specs=[
            pl.BlockSpec((blk, D), lambda i: (i, 0)),
            pl.BlockSpec((blk, D), lambda i: (i, 0)),
            pl.BlockSpec((blk, NC), lambda i: (i, 0)),
            pl.BlockSpec((D, D), lambda i: (0, 0)),
            pl.BlockSpec((1, D), lambda i: (0, 0)),
            pl.BlockSpec((D, D), lambda i: (0, 0)),
            pl.BlockSpec((1, D), lambda i: (0, 0)),
        ],
        out_specs=pl.BlockSpec((blk, D), lambda i: (i, 0)),
        out_shape=jax.ShapeDtypeStruct((N, D), jnp.float32),
    )(msgpart[0], msgpart[1], denom.T, wa, ba.reshape(1, D), wo,
      bo.reshape(1, D))


def kernel(node_features, adj_indices, adj_values, adj_dense_shape,
           attention_weights, transform_weights, transform_bias,
           activation_weights, activation_bias, output_weights, output_bias):
    x = node_features[0]
    att = attention_weights[0, :, 0]
    src = adj_indices[:, 1]
    dst = adj_indices[:, 2]
    pad = EP - E
    src_p = jnp.pad(src, (0, pad))
    dst_p = jnp.pad(dst, (0, pad))
    adj_p = jnp.pad(adj_values, (0, pad))
    adj_b = lax.bitcast_convert_type(adj_p, jnp.int32)
    pidx = jnp.stack([src_p, dst_p, adj_b], axis=0)
    pidx_a = pidx.reshape(3, NW * NCH, CHUNK).transpose(1, 0, 2)
    pidx_c = pidx[:2, :EP_C].reshape(2, EP_C // CHUNK_C, CHUNK_C)
    pidx_c = pidx_c.transpose(1, 0, 2)

    h, hl = _transform(x, transform_weights, transform_bias)

    def packb(t):
        tb = t.astype(jnp.bfloat16).reshape(N, D // 32, 2, 16)
        tb = tb.transpose(0, 1, 3, 2)
        return lax.bitcast_convert_type(tb, jnp.int32).reshape(N, D // 2)

    hlb = packb(hl)
    hb = packb(h)
    scores, mpart = _stage_a(hlb, hb, pidx_a, att)
    m = _merge_max(mpart).reshape(N)
    denom, msgpart = _stage_c(h, pidx_c, scores, m)
    denom = denom[:, :N]
    msgpart = msgpart[:, :N, :]
    out = _stage_d(msgpart, denom, activation_weights, activation_bias,
                   output_weights, output_bias)
    return out[None, :, :]


# speedup vs baseline: 1.0249x; 1.0039x over previous
import jax
import jax.numpy as jnp
from jax import lax
from jax.experimental import pallas as pl
from jax.experimental.pallas import tpu as pltpu
from jax.experimental.pallas import tpu_sc as plsc

N = 10000
D = 128
E = 320000
NC = 2
NS = 16
NW = NC * NS
CHUNK = 128
NCH = 80
EPT = NCH * CHUNK
EP = NW * EPT
NP = 10240
CHUNK_C = 96
NCHA0, NCHA1 = 112, 48
NCHC0, NCHC1 = 146, 66
EP_C = NS * (NCHC0 + NCHC1) * CHUNK_C
NEG = -1e30


def _iota16():
    return lax.iota(jnp.int32, 16)


def _lanesum(v):
    i = _iota16()
    for sh in (8, 4, 2, 1):
        v = v + v.at[i ^ sh].get(mode="promise_in_bounds")
    return v


def _s0_body(x_ref, w_ref, b_ref, h_ref, hl_ref):
    h = jnp.dot(x_ref[...], w_ref[...], preferred_element_type=jnp.float32)
    h = h + b_ref[...]
    h_ref[...] = h
    col = lax.broadcasted_iota(jnp.int32, h.shape, 1)
    hl_ref[...] = jnp.where(col == 0, -h, h)


def _transform(x, wt, bt):
    blk = 1000
    return pl.pallas_call(
        _s0_body,
        grid=(N // blk,),
        in_specs=[
            pl.BlockSpec((blk, D), lambda i: (i, 0)),
            pl.BlockSpec((D, D), lambda i: (0, 0)),
            pl.BlockSpec((1, D), lambda i: (0, 0)),
        ],
        out_specs=[
            pl.BlockSpec((blk, D), lambda i: (i, 0)),
            pl.BlockSpec((blk, D), lambda i: (i, 0)),
        ],
        out_shape=[
            jax.ShapeDtypeStruct((N, D), jnp.float32),
            jax.ShapeDtypeStruct((N, D), jnp.float32),
        ],
    )(x, wt, bt.reshape(1, D))


def _stage_a_body(hl_hbm, h_hbm, pidx_hbm, att_hbm,
                  scores_hbm, mpart_hbm,
                  att_v, m_v, pidx0, pidx1, hsl0, hsl1, hd0, hd1, sc_v,
                  sema0, sema1, semb0, semb1):
    cid = lax.axis_index("c")
    sid = lax.axis_index("s")
    wid = cid * NS + sid
    nch = jnp.where(cid == 0, NCHA0, NCHA1)
    cbase = jnp.where(cid == 0, sid * NCHA0, NS * NCHA0 + sid * NCHA1)
    pidx = (pidx0, pidx1)
    hsl = (hsl0, hsl1)
    hd = (hd0, hd1)
    sema = (sema0, sema1)
    semb = (semb0, semb1)
    pltpu.sync_copy(att_hbm, att_v)

    def init_m(i, _):
        m_v[pl.ds(i * 16, 16)] = jnp.full((16,), NEG, jnp.float32)
        return 0
    lax.fori_loop(0, N // 16, init_m, 0)

    def fetch(nci, q):
        pltpu.sync_copy(pidx_hbm.at[cbase + nci], pidx[q])
        pltpu.async_copy(hl_hbm.at[pidx[q].at[0]], hsl[q], sema[q])
        pltpu.async_copy(h_hbm.at[pidx[q].at[1]], hd[q], semb[q])

    def compute(ci, p):
        base = (cbase + ci) * CHUNK

        def group_body(g, _):
            gbase = g * 16
            lor = jnp.zeros((16,), jnp.float32)
            for j in range(16):
                acc = jnp.zeros((16,), jnp.float32)
                for k in range(D // 32):
                    a = plsc.bitcast(hsl[p][gbase + j, pl.ds(k * 16, 16)],
                                     jnp.bfloat16)
                    b = plsc.bitcast(hd[p][gbase + j, pl.ds(k * 16, 16)],
                                     jnp.bfloat16)
                    a0, a1 = plsc.unpack(a, format=plsc.PackFormat.INTERLEAVED)
                    b0, b1 = plsc.unpack(b, format=plsc.PackFormat.INTERLEAVED)
                    acc = acc + a0 * b0 + a1 * b1
                dj = _lanesum(acc)
                lor = jnp.where(_iota16() == j, dj, lor)

            src_g = pidx[p][0, pl.ds(gbase, 16)]
            dst_g = pidx[p][1, pl.ds(gbase, 16)]
            adj_g = plsc.bitcast(pidx[p][2, pl.ds(gbase, 16)], jnp.float32)
            att_s = plsc.load_gather(att_v, [src_g])
            att_d = plsc.load_gather(att_v, [dst_g])
            s = adj_g * lor + att_s + att_d
            s = jnp.where(s >= 0.0, s, 0.2 * s)
            eid = base + gbase + _iota16()
            s = jnp.where(eid < E, s, NEG)
            sc_v[pl.ds(gbase, 16)] = s

            cur = plsc.load_gather(m_v, [dst_g])

            def cond(carry):
                return carry[1]

            def body(carry):
                c2 = plsc.load_gather(m_v, [dst_g])
                plsc.store_scatter(m_v, [dst_g], jnp.maximum(c2, s),
                                   mask=s > c2)
                c3 = plsc.load_gather(m_v, [dst_g])
                return (c3, jnp.any(s > c3))
            lax.while_loop(cond, body, (cur, jnp.any(s > cur)))
            return 0
        lax.fori_loop(0, CHUNK // 16, group_body, 0)
        pltpu.sync_copy(sc_v, scores_hbm.at[pl.ds(base, CHUNK)])

    fetch(0, 0)

    def pair_body(t, _):
        for half in (0, 1):
            pb = half
            qb = 1 - half
            ci = t * 2 + half
            nci = ci + 1

            @pl.when(nci < nch)
            def _():
                fetch(nci, qb)
            pltpu.make_async_copy(
                hl_hbm.at[pidx[pb].at[0]], hsl[pb], sema[pb]).wait()
            pltpu.make_async_copy(
                h_hbm.at[pidx[pb].at[1]], hd[pb], semb[pb]).wait()
            compute(ci, pb)
        return 0
    lax.fori_loop(0, nch // 2, pair_body, 0)
    pltpu.sync_copy(m_v, mpart_hbm.at[wid])


def _stage_a(hl, h, pidx, att):
    mesh = plsc.VectorSubcoreMesh(core_axis_name="c", subcore_axis_name="s")
    return pl.kernel(
        _stage_a_body,
        out_type=[
            jax.ShapeDtypeStruct((EP,), jnp.float32),
            jax.ShapeDtypeStruct((NW, N), jnp.float32),
        ],
        mesh=mesh,
        compiler_params=pltpu.CompilerParams(needs_layout_passes=False,
                                             use_tc_tiling_on_sc=False),
        scratch_types=[
            pltpu.VMEM((N,), jnp.float32),
            pltpu.VMEM((N,), jnp.float32),
            pltpu.VMEM((3, CHUNK), jnp.int32),
            pltpu.VMEM((3, CHUNK), jnp.int32),
            pltpu.VMEM((CHUNK, D // 2), jnp.int32),
            pltpu.VMEM((CHUNK, D // 2), jnp.int32),
            pltpu.VMEM((CHUNK, D // 2), jnp.int32),
            pltpu.VMEM((CHUNK, D // 2), jnp.int32),
            pltpu.VMEM((CHUNK,), jnp.float32),
            pltpu.SemaphoreType.DMA,
            pltpu.SemaphoreType.DMA,
            pltpu.SemaphoreType.DMA,
            pltpu.SemaphoreType.DMA,
        ],
    )(hl, h, pidx, att)


def _merge_max_body(mp_ref, m_ref):
    m_ref[...] = jnp.max(mp_ref[...], axis=0, keepdims=True)


def _merge_max(mpart):
    return pl.pallas_call(
        _merge_max_body,
        out_shape=jax.ShapeDtypeStruct((1, N), jnp.float32),
    )(mpart)


def _stage_c_body(h_hbm, pidx_hbm, scores_hbm, m_hbm,
                  denom_hbm, msgpart_hbm,
                  m_v, pidx0, pidx1, sc0, sc1, e_v,
                  g0, g1, msg_sh, den_sh,
                  semg0, semg1):
    cid = lax.axis_index("c")
    sid = lax.axis_index("s")
    wid = cid * NS + sid
    nch = jnp.where(cid == 0, NCHC0, NCHC1)
    cbase = jnp.where(cid == 0, sid * NCHC0, NS * NCHC0 + sid * NCHC1)
    pidx = (pidx0, pidx1)
    scv = (sc0, sc1)
    gv = (g0, g1)
    semg = (semg0, semg1)
    pltpu.sync_copy(m_hbm, m_v)

    def init_f(i, _):
        for k in range(D // 16):
            g0[i, pl.ds(k * 16, 16)] = jnp.zeros((16,), jnp.float32)
        return 0
    lax.fori_loop(0, CHUNK_C, init_f, 0)

    def init_e(i, _):
        e_v[pl.ds(i * 16, 16)] = jnp.zeros((16,), jnp.float32)
        return 0
    lax.fori_loop(0, CHUNK_C // 16, init_e, 0)
    rows_per_tile = NP // NS
    r0 = sid * rows_per_tile
    for z in range(6):
        pltpu.sync_copy(g0, msg_sh.at[pl.ds(r0 + z * CHUNK_C, CHUNK_C)])
        pltpu.sync_copy(e_v, den_sh.at[pl.ds(r0 + z * CHUNK_C, CHUNK_C)])
    pltpu.sync_copy(g0.at[pl.ds(0, 64)],
                    msg_sh.at[pl.ds(r0 + 6 * CHUNK_C, 64)])
    pltpu.sync_copy(e_v.at[pl.ds(0, 64)],
                    den_sh.at[pl.ds(r0 + 6 * CHUNK_C, 64)])
    plsc.subcore_barrier()

    def fetch(nci, q):
        pltpu.sync_copy(pidx_hbm.at[cbase + nci], pidx[q])
        pltpu.sync_copy(
            scores_hbm.at[pl.ds((cbase + nci) * CHUNK_C, CHUNK_C)], scv[q])
        pltpu.async_copy(h_hbm.at[pidx[q].at[0]], gv[q], semg[q])

    def compute(ci, p):
        def group_body(g, _):
            gbase = g * 16
            s = scv[p][pl.ds(gbase, 16)]
            dst_g = pidx[p][1, pl.ds(gbase, 16)]
            m_d = plsc.load_gather(m_v, [dst_g])
            e = jnp.where(s < -5e29, 0.0, jnp.exp(s - m_d))
            e_v[pl.ds(gbase, 16)] = e
            return 0
        lax.fori_loop(0, CHUNK_C // 16, group_body, 0)
        pltpu.sync_copy(e_v, den_sh.at[pidx[p].at[1]], add=True)

        def scale_body(g, _):
            e16 = e_v[pl.ds(g * 16, 16)]
            for j in range(16):
                a = e16[j]
                r = g * 16 + j
                for k in range(D // 16):
                    gv[p][r, pl.ds(k * 16, 16)] = (
                        gv[p][r, pl.ds(k * 16, 16)] * a)
            return 0
        lax.fori_loop(0, CHUNK_C // 16, scale_body, 0)
        pltpu.sync_copy(gv[p], msg_sh.at[pidx[p].at[1]], add=True)

    fetch(0, 0)

    def pair_body(t, _):
        for half in (0, 1):
            pb = half
            qb = 1 - half
            ci = t * 2 + half
            nci = ci + 1

            @pl.when(nci < nch)
            def _():
                fetch(nci, qb)
            pltpu.make_async_copy(
                h_hbm.at[pidx[pb].at[0]], gv[pb], semg[pb]).wait()
            compute(ci, pb)
        return 0
    lax.fori_loop(0, nch // 2, pair_body, 0)
    plsc.subcore_barrier()
    pltpu.sync_copy(den_sh.at[pl.ds(r0, rows_per_tile)],
                    denom_hbm.at[cid, pl.ds(r0, rows_per_tile)])
    pltpu.sync_copy(msg_sh.at[pl.ds(r0, rows_per_tile)],
                    msgpart_hbm.at[cid, pl.ds(r0, rows_per_tile)])


def _stage_c(h, pidx, scores, m):
    mesh = plsc.VectorSubcoreMesh(core_axis_name="c", subcore_axis_name="s")
    return pl.kernel(
        _stage_c_body,
        out_type=[
            jax.ShapeDtypeStruct((NC, NP), jnp.float32),
            jax.ShapeDtypeStruct((NC, NP, D), jnp.float32),
        ],
        mesh=mesh,
        compiler_params=pltpu.CompilerParams(needs_layout_passes=False,
                                             use_tc_tiling_on_sc=False),
        scratch_types=[
            pltpu.VMEM((N,), jnp.float32),
            pltpu.VMEM((2, CHUNK_C), jnp.int32),
            pltpu.VMEM((2, CHUNK_C), jnp.int32),
            pltpu.VMEM((CHUNK_C,), jnp.float32),
            pltpu.VMEM((CHUNK_C,), jnp.float32),
            pltpu.VMEM((CHUNK_C,), jnp.float32),
            pltpu.VMEM((CHUNK_C, D), jnp.float32),
            pltpu.VMEM((CHUNK_C, D), jnp.float32),
            pltpu.VMEM_SHARED((NP, D), jnp.float32),
            pltpu.VMEM_SHARED((NP,), jnp.float32),
            pltpu.SemaphoreType.DMA,
            pltpu.SemaphoreType.DMA,
        ],
    )(h, pidx, scores, m)


def _d_body(msg0_ref, msg1_ref, den_ref, wa_ref, ba_ref, wo_ref, bo_ref,
            out_ref):
    dsum = jnp.sum(den_ref[...], axis=1, keepdims=True) + 1e-9
    msg = (msg0_ref[...] + msg1_ref[...]) * (1.0 / dsum)
    act = jnp.tanh(
        jnp.dot(msg, wa_ref[...], preferred_element_type=jnp.float32)
        + ba_ref[...])
    out_ref[...] = (
        jnp.dot(act, wo_ref[...], preferred_element_type=jnp.float32)
        + bo_ref[...])


def _stage_d(msgpart, denom, wa, ba, wo, bo):
    blk = 1000
    return pl.pallas_call(
        _d_body,
        grid=(N // blk,),
        in_specs=[
            pl.BlockSpec((blk, D), lambda i: (i, 0)),
            pl.BlockSpec((blk, D), lambda i: (i, 0)),
            pl.BlockSpec((blk, NC), lambda i: (i, 0)),
            pl.BlockSpec((D, D), lambda i: (0, 0)),
            pl.BlockSpec((1, D), lambda i: (0, 0)),
            pl.BlockSpec((D, D), lambda i: (0, 0)),
            pl.BlockSpec((1, D), lambda i: (0, 0)),
        ],
        out_specs=pl.BlockSpec((blk, D), lambda i: (i, 0)),
        out_shape=jax.ShapeDtypeStruct((N, D), jnp.float32),
    )(msgpart[0], msgpart[1], denom.T, wa, ba.reshape(1, D), wo,
      bo.reshape(1, D))


def kernel(node_features, adj_indices, adj_values, adj_dense_shape,
           attention_weights, transform_weights, transform_bias,
           activation_weights, activation_bias, output_weights, output_bias):
    x = node_features[0]
    att = attention_weights[0, :, 0]
    src = adj_indices[:, 1]
    dst = adj_indices[:, 2]
    pad = EP - E
    src_p = jnp.pad(src, (0, pad))
    dst_p = jnp.pad(dst, (0, pad))
    adj_p = jnp.pad(adj_values, (0, pad))
    adj_b = lax.bitcast_convert_type(adj_p, jnp.int32)
    pidx = jnp.stack([src_p, dst_p, adj_b], axis=0)
    pidx_a = pidx.reshape(3, NW * NCH, CHUNK).transpose(1, 0, 2)
    pidx_c = pidx[:2, :EP_C].reshape(2, EP_C // CHUNK_C, CHUNK_C)
    pidx_c = pidx_c.transpose(1, 0, 2)

    h, hl = _transform(x, transform_weights, transform_bias)

    def packb(t):
        tb = t.astype(jnp.bfloat16).reshape(N, D // 32, 2, 16)
        tb = tb.transpose(0, 1, 3, 2)
        return lax.bitcast_convert_type(tb, jnp.int32).reshape(N, D // 2)

    hlb = packb(hl)
    hb = packb(h)
    scores, mpart = _stage_a(hlb, hb, pidx_a, att)
    m = _merge_max(mpart).reshape(N)
    denom, msgpart = _stage_c(h, pidx_c, scores, m)
    denom = denom[:, :N]
    msgpart = msgpart[:, :N, :]
    out = _stage_d(msgpart, denom, activation_weights, activation_bias,
                   output_weights, output_bias)
    return out[None, :, :]
